# Initial kernel scaffold; baseline (speedup 1.0000x reference)
import jax
import jax.numpy as jnp
from jax.experimental import pallas as pl


def kernel(x, target):
    raise NotImplementedError("write your pallas kernel here")


# speedup vs baseline: 5.2787x; 5.2787x over previous
import math

import numpy as np
import jax
import jax.numpy as jnp
from jax.experimental import pallas as pl
from jax.experimental.pallas import tpu as pltpu

_V = 32000
_N = 2048
_S = float(np.float32(0.1 / (_V - 2)))
_C = 0.9
_CONST_PER_ROW = (_V - 2) * _S * math.log(_S) + _C * math.log(_C)

_BR = 256
_BC = 3200
_NRB = _N // _BR
_NCB = _V // _BC


def _body(t_ref, x_ref, o_ref, acc_ref):
    i = pl.program_id(0)
    j = pl.program_id(1)

    @pl.when((i == 0) & (j == 0))
    def _():
        acc_ref[0] = 0.0
        acc_ref[1] = 0.0
        acc_ref[2] = 0.0

    xb = x_ref[...]
    t = t_ref[0, 0, :]
    col = jax.lax.broadcasted_iota(jnp.int32, (_BR, _BC), 1) + j * _BC
    xz = jnp.where(col == 0, 0.0, xb)
    rowp = jnp.sum(xz, axis=1)
    rowp = jnp.where(t == 0, 0.0, rowp)
    acc_ref[0] += jnp.sum(rowp)
    match = (col == t[:, None]) & (t[:, None] != 0)
    acc_ref[1] += jnp.sum(jnp.where(match, xb, 0.0))

    @pl.when(j == 0)
    def _():
        acc_ref[2] += jnp.sum((t != 0).astype(jnp.float32))

    @pl.when((i == _NRB - 1) & (j == _NCB - 1))
    def _():
        o_ref[0, 0] = (acc_ref[2] * _CONST_PER_ROW
                       - (_S * acc_ref[0] + (_C - _S) * acc_ref[1])) / _N


def kernel(x, target):
    t3 = target.astype(jnp.int32).reshape(_NRB, 1, _BR)
    out = pl.pallas_call(
        _body,
        grid=(_NRB, _NCB),
        in_specs=[
            pl.BlockSpec((1, 1, _BR), lambda i, j: (i, 0, 0)),
            pl.BlockSpec((_BR, _BC), lambda i, j: (i, j)),
        ],
        out_specs=pl.BlockSpec(memory_space=pltpu.SMEM),
        out_shape=jax.ShapeDtypeStruct((1, 1), jnp.float32),
        scratch_shapes=[pltpu.SMEM((3,), jnp.float32)],
    )(t3, x)
    return out[0, 0]
